# baseline (speedup 1.0000x reference)
import functools

import jax
import jax.numpy as jnp
from jax import lax
from jax.experimental import pallas as pl
from jax.experimental.pallas import tpu as pltpu
from jax.experimental.pallas import tpu_sc as plsc

_N = 10000
_E = 160000
_D = 256
_G = 64
_DO = 128

_NT = 32
_NPT = 320
_NPAD = _NT * _NPT
_EPT = _E // _NT
_EGRP = _EPT // 16
_ETAIL = _EPT - _EGRP * 16
_CAPR = 5888
_KG = 160
_NV = _D // 16


def _splat(x):
    return jnp.full((16,), x, jnp.int32)


def _permute16(v, idx):
    return lax.gather(
        v, idx[:, None],
        dimension_numbers=lax.GatherDimensionNumbers(
            offset_dims=(), collapsed_slice_dims=(0,), start_index_map=(0,)),
        slice_sizes=(1,),
        mode=lax.GatherScatterMode.PROMISE_IN_BOUNDS)


def _filter_body(srce, dste, ewe, osrc, odst, oew, otab,
                 es, ed, ew_, bsrc, bdst, bew, tabst, smem):
    cid = lax.axis_index("c")
    sid = lax.axis_index("s")
    wid = sid * 2 + cid
    ebase = wid * _EPT

    pltpu.sync_copy(srce.at[pl.ds(ebase, _EPT)], es.at[pl.ds(0, _EPT)])
    pltpu.sync_copy(dste.at[pl.ds(ebase, _EPT)], ed.at[pl.ds(0, _EPT)])
    pltpu.sync_copy(ewe.at[pl.ds(ebase, _EPT)], ew_.at[pl.ds(0, _EPT)])

    zi = jnp.zeros((16,), jnp.int32)

    def _z(i, c):
        bsrc[pl.ds(i * 16, 16)] = zi
        return c

    lax.fori_loop(0, _CAPR // 16, _z, 0)

    for o in range(32):
        smem[o] = 0

    def _cnt_grp(g, c):
        dv16 = ed[pl.ds(g * 16, 16)]
        ov16 = (dv16 * 6554) >> 21
        for k in range(16):
            o = ov16[k]
            smem[o] = smem[o] + 1
        return c

    lax.fori_loop(0, _EGRP, _cnt_grp, 0)
    dvt = ed[pl.ds(_EGRP * 16, 16)]
    ovt = (dvt * 6554) >> 21
    for k in range(_ETAIL):
        o = ovt[k]
        smem[o] = smem[o] + 1

    def _off(o, t):
        smem[32 + o] = t
        smem[64 + o] = t
        c = smem[o]
        return t + ((c + 7) & ~7) + 16

    lax.fori_loop(0, 32, _off, jnp.int32(0))

    def _put(e_base, sv16, dv16, wv16, ov16, k):
        o = ov16[k]
        p = smem[64 + o]
        bsrc[pl.ds(p, 16)] = _splat(sv16[k])
        bdst[pl.ds(p, 16)] = _splat(dv16[k])
        bew[pl.ds(p, 16)] = jnp.full((16,), wv16[k], jnp.float32)
        smem[64 + o] = p + 1

    def _put_grp(g, c):
        gb = g * 16
        sv16 = es[pl.ds(gb, 16)]
        dv16 = ed[pl.ds(gb, 16)]
        wv16 = ew_[pl.ds(gb, 16)]
        ov16 = (dv16 * 6554) >> 21
        for k in range(16):
            _put(gb, sv16, dv16, wv16, ov16, k)
        return c

    lax.fori_loop(0, _EGRP, _put_grp, 0)
    svt = es[pl.ds(_EGRP * 16, 16)]
    wvt = ew_[pl.ds(_EGRP * 16, 16)]
    for k in range(_ETAIL):
        _put(0, svt, dvt, wvt, ovt, k)

    lane = lax.iota(jnp.int32, 16)
    t0 = jnp.zeros((16,), jnp.int32)
    t1 = jnp.zeros((16,), jnp.int32)
    t2 = jnp.zeros((16,), jnp.int32)
    t3 = jnp.zeros((16,), jnp.int32)
    for o in range(16):
        t0 = jnp.where(lane == o, _splat(smem[32 + o]), t0)
        t1 = jnp.where(lane == o, _splat(smem[48 + o]), t1)
        t2 = jnp.where(lane == o, _splat(smem[o]), t2)
        t3 = jnp.where(lane == o, _splat(smem[16 + o]), t3)
    tabst[pl.ds(0, 16)] = t0
    tabst[pl.ds(16, 16)] = t1
    tabst[pl.ds(32, 16)] = t2
    tabst[pl.ds(48, 16)] = t3

    rbase = wid * _CAPR
    pltpu.sync_copy(bsrc, osrc.at[pl.ds(rbase, _CAPR)])
    pltpu.sync_copy(bdst, odst.at[pl.ds(rbase, _CAPR)])
    pltpu.sync_copy(bew, oew.at[pl.ds(rbase, _CAPR)])
    pltpu.sync_copy(tabst, otab.at[pl.ds(wid * 64, 64)])


_sc_filter = functools.partial(
    pl.kernel,
    out_type=(jax.ShapeDtypeStruct((_NT * _CAPR,), jnp.int32),
              jax.ShapeDtypeStruct((_NT * _CAPR,), jnp.int32),
              jax.ShapeDtypeStruct((_NT * _CAPR,), jnp.float32),
              jax.ShapeDtypeStruct((_NT * 64,), jnp.int32)),
    mesh=plsc.VectorSubcoreMesh(core_axis_name="c", subcore_axis_name="s"),
    scratch_types=[
        pltpu.VMEM((_EPT + 24,), jnp.int32),
        pltpu.VMEM((_EPT + 24,), jnp.int32),
        pltpu.VMEM((_EPT + 24,), jnp.float32),
        pltpu.VMEM((_CAPR,), jnp.int32),
        pltpu.VMEM((_CAPR,), jnp.int32),
        pltpu.VMEM((_CAPR,), jnp.float32),
        pltpu.VMEM((64,), jnp.int32),
        pltpu.SMEM((96,), jnp.int32),
    ],
)(_filter_body)


def _acc_body(hl, lsrc, ldst, lew, ltab, out,
              acc, ctab, st_src, st_dst, st_ew, rows, sem):
    cid = lax.axis_index("c")
    sid = lax.axis_index("s")
    wid = sid * 2 + cid
    base = wid * _NPT

    zf = jnp.zeros((16,), jnp.float32)

    def _zero_row(i, c):
        for v in range(_NV):
            acc[i, pl.ds(v * 16, 16)] = zf
        return c

    lax.fori_loop(0, _NPT, _zero_row, 0)

    pltpu.sync_copy(ltab, ctab.at[pl.ds(0, _NT * 64)])

    def _scanner(s, c):
        sb = s * 64
        off = pl.multiple_of(ctab[pl.ds(sb + wid, 16)][0], 8)
        cnt = ctab[pl.ds(sb + 32 + wid, 16)][0]
        nch = ((cnt + _KG - 1) * 13108) >> 21
        rb = s * _CAPR + off

        def _chunk(ch, c2):
            cb = rb + ch * _KG
            c_a = pltpu.async_copy(lsrc.at[pl.ds(cb, _KG)], st_src, sem)
            c_b = pltpu.async_copy(ldst.at[pl.ds(cb, _KG)],
                                   st_dst.at[pl.ds(0, _KG)], sem)
            c_c = pltpu.async_copy(lew.at[pl.ds(cb, _KG)],
                                   st_ew.at[pl.ds(0, _KG)], sem)
            c_a.wait()
            c_b.wait()
            c_c.wait()
            pltpu.async_copy(hl.at[st_src], rows, sem).wait()
            nval = jnp.minimum(cnt - ch * _KG, _KG)
            nsub = nval >> 4

            def _sub(g, c3):
                gb = g * 16
                dv16 = st_dst[pl.ds(gb, 16)]
                wv16 = st_ew[pl.ds(gb, 16)]
                for k in range(16):
                    li = dv16[k] - base
                    w = wv16[k]
                    for v in range(_NV):
                        val = rows[gb + k, pl.ds(v * 16, 16)] * w
                        plsc.addupdate(acc.at[li, pl.ds(v * 16, 16)], val)
                return c3

            lax.fori_loop(0, nsub, _sub, 0)

            def _edge(e, c3):
                li = st_dst[pl.ds(e, 16)][0] - base
                w = st_ew[pl.ds(e, 16)][0]
                for v in range(_NV):
                    val = rows[e, pl.ds(v * 16, 16)] * w
                    plsc.addupdate(acc.at[li, pl.ds(v * 16, 16)], val)
                return c3

            lax.fori_loop(nsub * 16, nval, _edge, 0)
            return c2

        lax.fori_loop(0, nch, _chunk, 0)
        return c

    lax.fori_loop(0, _NT, _scanner, 0)

    pltpu.sync_copy(acc, out.at[pl.ds(base, _NPT)])


_sc_acc = functools.partial(
    pl.kernel,
    out_type=jax.ShapeDtypeStruct((_NPAD, _D), jnp.float32),
    mesh=plsc.VectorSubcoreMesh(core_axis_name="c", subcore_axis_name="s"),
    scratch_types=[
        pltpu.VMEM((_NPT, _D), jnp.float32),
        pltpu.VMEM((_NT * 64 + 16,), jnp.int32),
        pltpu.VMEM((_KG,), jnp.int32),
        pltpu.VMEM((_KG + 16,), jnp.int32),
        pltpu.VMEM((_KG + 16,), jnp.float32),
        pltpu.VMEM((_KG, _D), jnp.float32),
        pltpu.SemaphoreType.DMA,
    ],
)(_acc_body)


def _mm_body(x_ref, w_ref, o_ref):
    o_ref[...] = jnp.dot(x_ref[...], w_ref[...],
                         preferred_element_type=jnp.float32)


def _bn_mm_body(agg_ref, b_ref, g_ref, be_ref, w_ref, h_ref, hl_ref):
    h = agg_ref[...] + b_ref[...]
    mu = jnp.mean(h, axis=0, keepdims=True)
    d = h - mu
    var = jnp.mean(d * d, axis=0, keepdims=True)
    hr = jnp.maximum(d * lax.rsqrt(var + 1e-5) * g_ref[...] + be_ref[...], 0.0)
    h_ref[...] = hr
    hl_ref[...] = jnp.dot(hr, w_ref[...], preferred_element_type=jnp.float32)


def _final_body(agg_ref, b_ref, g_ref, be_ref, x_ref, h1_ref, bf_ref,
                wp0_ref, bp0_ref, wp1_ref, bp1_ref, wp2_ref, bp2_ref, o_ref):
    h = agg_ref[...] + b_ref[...]
    mu = jnp.mean(h, axis=0, keepdims=True)
    d = h - mu
    var = jnp.mean(d * d, axis=0, keepdims=True)
    h2 = jnp.maximum(d * lax.rsqrt(var + 1e-5) * g_ref[...] + be_ref[...], 0.0)

    ids = lax.broadcasted_iota(jnp.int32, (_G, _N), 0)
    oh = (bf_ref[...] == ids).astype(jnp.float32)
    counts = jnp.maximum(jnp.sum(oh, axis=1, keepdims=True), 1.0)
    p0 = jnp.dot(oh, x_ref[...], preferred_element_type=jnp.float32) / counts
    p1 = jnp.dot(oh, h1_ref[...], preferred_element_type=jnp.float32) / counts
    p2 = jnp.dot(oh, h2, preferred_element_type=jnp.float32) / counts
    r = (jnp.dot(p0, wp0_ref[...], preferred_element_type=jnp.float32)
         + bp0_ref[...]
         + jnp.dot(p1, wp1_ref[...], preferred_element_type=jnp.float32)
         + bp1_ref[...]
         + jnp.dot(p2, wp2_ref[...], preferred_element_type=jnp.float32)
         + bp2_ref[...])
    o_ref[...] = jax.nn.sigmoid(r)


def kernel(x, edge_index, edge_weight, batch,
           W0, b0, W1, b1, g0, be0, g1, be1,
           Wp0, bp0, Wp1, bp1, Wp2, bp2):
    src = edge_index[0]
    dst = edge_index[1]
    batch_f = batch.reshape(1, _N)

    b0r, g0r, be0r = (v.reshape(1, _D) for v in (b0, g0, be0))
    b1r, g1r, be1r = (v.reshape(1, _D) for v in (b1, g1, be1))
    bp0r, bp1r, bp2r = (v.reshape(1, _DO) for v in (bp0, bp1, bp2))

    lsrc, ldst, lew, ltab = _sc_filter(src, dst, edge_weight)

    hl0 = pl.pallas_call(
        _mm_body,
        out_shape=jax.ShapeDtypeStruct((_N, _D), jnp.float32),
    )(x, W0)

    agg0 = _sc_acc(hl0, lsrc, ldst, lew, ltab)[:_N]

    h1, hl1 = pl.pallas_call(
        _bn_mm_body,
        out_shape=(jax.ShapeDtypeStruct((_N, _D), jnp.float32),
                   jax.ShapeDtypeStruct((_N, _D), jnp.float32)),
    )(agg0, b0r, g0r, be0r, W1)

    agg1 = _sc_acc(hl1, lsrc, ldst, lew, ltab)[:_N]

    out = pl.pallas_call(
        _final_body,
        out_shape=jax.ShapeDtypeStruct((_G, _DO), jnp.float32),
    )(agg1, b1r, g1r, be1r, x, h1, batch_f,
      Wp0, bp0r, Wp1, bp1r, Wp2, bp2r)
    return out


# speedup vs baseline: 1.1958x; 1.1958x over previous
import functools

import jax
import jax.numpy as jnp
from jax import lax
from jax.experimental import pallas as pl
from jax.experimental.pallas import tpu as pltpu
from jax.experimental.pallas import tpu_sc as plsc

_N = 10000
_E = 160000
_D = 256
_G = 64
_DO = 128

_NT = 32
_NPT = 320
_NPAD = _NT * _NPT
_EPT = _E // _NT
_EGRP = _EPT // 16
_ETAIL = _EPT - _EGRP * 16
_CAPR = 5888
_KG = 88
_NV = _D // 16


def _splat(x):
    return jnp.full((16,), x, jnp.int32)


def _permute16(v, idx):
    return lax.gather(
        v, idx[:, None],
        dimension_numbers=lax.GatherDimensionNumbers(
            offset_dims=(), collapsed_slice_dims=(0,), start_index_map=(0,)),
        slice_sizes=(1,),
        mode=lax.GatherScatterMode.PROMISE_IN_BOUNDS)


def _filter_body(srce, dste, ewe, osrc, odst, oew, otab,
                 es, ed, ew_, bsrc, bdst, bew, tabst, smem):
    cid = lax.axis_index("c")
    sid = lax.axis_index("s")
    wid = sid * 2 + cid
    ebase = wid * _EPT

    pltpu.sync_copy(srce.at[pl.ds(ebase, _EPT)], es.at[pl.ds(0, _EPT)])
    pltpu.sync_copy(dste.at[pl.ds(ebase, _EPT)], ed.at[pl.ds(0, _EPT)])
    pltpu.sync_copy(ewe.at[pl.ds(ebase, _EPT)], ew_.at[pl.ds(0, _EPT)])

    zi = jnp.zeros((16,), jnp.int32)

    def _z(i, c):
        bsrc[pl.ds(i * 16, 16)] = zi
        return c

    lax.fori_loop(0, _CAPR // 16, _z, 0)

    for o in range(32):
        smem[o] = 0

    def _cnt_grp(g, c):
        dv16 = ed[pl.ds(g * 16, 16)]
        ov16 = (dv16 * 6554) >> 21
        for k in range(16):
            o = ov16[k]
            smem[o] = smem[o] + 1
        return c

    lax.fori_loop(0, _EGRP, _cnt_grp, 0)
    dvt = ed[pl.ds(_EGRP * 16, 16)]
    ovt = (dvt * 6554) >> 21
    for k in range(_ETAIL):
        o = ovt[k]
        smem[o] = smem[o] + 1

    def _off(o, t):
        smem[32 + o] = t
        smem[64 + o] = t
        c = smem[o]
        return t + ((c + 7) & ~7) + 16

    lax.fori_loop(0, 32, _off, jnp.int32(0))

    def _put(e_base, sv16, dv16, wv16, ov16, k):
        o = ov16[k]
        p = smem[64 + o]
        bsrc[pl.ds(p, 16)] = _splat(sv16[k])
        bdst[pl.ds(p, 16)] = _splat(dv16[k])
        bew[pl.ds(p, 16)] = jnp.full((16,), wv16[k], jnp.float32)
        smem[64 + o] = p + 1

    def _put_grp(g, c):
        gb = g * 16
        sv16 = es[pl.ds(gb, 16)]
        dv16 = ed[pl.ds(gb, 16)]
        wv16 = ew_[pl.ds(gb, 16)]
        ov16 = (dv16 * 6554) >> 21
        for k in range(16):
            _put(gb, sv16, dv16, wv16, ov16, k)
        return c

    lax.fori_loop(0, _EGRP, _put_grp, 0)
    svt = es[pl.ds(_EGRP * 16, 16)]
    wvt = ew_[pl.ds(_EGRP * 16, 16)]
    for k in range(_ETAIL):
        _put(0, svt, dvt, wvt, ovt, k)

    lane = lax.iota(jnp.int32, 16)
    t0 = jnp.zeros((16,), jnp.int32)
    t1 = jnp.zeros((16,), jnp.int32)
    t2 = jnp.zeros((16,), jnp.int32)
    t3 = jnp.zeros((16,), jnp.int32)
    for o in range(16):
        t0 = jnp.where(lane == o, _splat(smem[32 + o]), t0)
        t1 = jnp.where(lane == o, _splat(smem[48 + o]), t1)
        t2 = jnp.where(lane == o, _splat(smem[o]), t2)
        t3 = jnp.where(lane == o, _splat(smem[16 + o]), t3)
    tabst[pl.ds(0, 16)] = t0
    tabst[pl.ds(16, 16)] = t1
    tabst[pl.ds(32, 16)] = t2
    tabst[pl.ds(48, 16)] = t3

    rbase = wid * _CAPR
    pltpu.sync_copy(bsrc, osrc.at[pl.ds(rbase, _CAPR)])
    pltpu.sync_copy(bdst, odst.at[pl.ds(rbase, _CAPR)])
    pltpu.sync_copy(bew, oew.at[pl.ds(rbase, _CAPR)])
    pltpu.sync_copy(tabst, otab.at[pl.ds(wid * 64, 64)])


_sc_filter = functools.partial(
    pl.kernel,
    out_type=(jax.ShapeDtypeStruct((_NT * _CAPR,), jnp.int32),
              jax.ShapeDtypeStruct((_NT * _CAPR,), jnp.int32),
              jax.ShapeDtypeStruct((_NT * _CAPR,), jnp.float32),
              jax.ShapeDtypeStruct((_NT * 64,), jnp.int32)),
    mesh=plsc.VectorSubcoreMesh(core_axis_name="c", subcore_axis_name="s"),
    scratch_types=[
        pltpu.VMEM((_EPT + 24,), jnp.int32),
        pltpu.VMEM((_EPT + 24,), jnp.int32),
        pltpu.VMEM((_EPT + 24,), jnp.float32),
        pltpu.VMEM((_CAPR,), jnp.int32),
        pltpu.VMEM((_CAPR,), jnp.int32),
        pltpu.VMEM((_CAPR,), jnp.float32),
        pltpu.VMEM((64,), jnp.int32),
        pltpu.SMEM((96,), jnp.int32),
    ],
)(_filter_body)


def _acc_body(hl, lsrc, ldst, lew, ltab, out,
              acc, ctab,
              sA_src, sA_dst, sA_ew, sB_src, sB_dst, sB_ew,
              sC_src, sC_dst, sC_ew, rowsA, rowsB,
              semSA, semSB, semSC, semGA, semGB):
    cid = lax.axis_index("c")
    sid = lax.axis_index("s")
    wid = sid * 2 + cid
    base = wid * _NPT

    zf = jnp.zeros((16,), jnp.float32)

    def _zero_row(i, c):
        for v in range(_NV):
            acc[i, pl.ds(v * 16, 16)] = zf
        return c

    lax.fori_loop(0, _NPT, _zero_row, 0)

    pltpu.sync_copy(ltab, ctab.at[pl.ds(0, _NT * 64)])

    def meta(s):
        sb = s * 64
        off = pl.multiple_of(ctab[pl.ds(sb + wid, 16)][0], 8)
        cnt = ctab[pl.ds(sb + 32 + wid, 16)][0]
        return off, cnt

    def fire_stage(s, ch, dsrc, ddst, dew, sem):
        sc = jnp.minimum(s, _NT - 1)
        off, _ = meta(sc)
        cb = sc * _CAPR + off + ch * _KG
        pltpu.async_copy(lsrc.at[pl.ds(cb, _KG)], dsrc, sem)
        pltpu.async_copy(ldst.at[pl.ds(cb, _KG)], ddst.at[pl.ds(0, _KG)], sem)
        pltpu.async_copy(lew.at[pl.ds(cb, _KG)], dew.at[pl.ds(0, _KG)], sem)

    def wait_stage(dsrc, ddst, dew, sem):
        pltpu.make_async_copy(lsrc.at[pl.ds(0, _KG)], dsrc, sem).wait()
        pltpu.make_async_copy(ldst.at[pl.ds(0, _KG)],
                              ddst.at[pl.ds(0, _KG)], sem).wait()
        pltpu.make_async_copy(lew.at[pl.ds(0, _KG)],
                              dew.at[pl.ds(0, _KG)], sem).wait()

    def fire_gather(dsrc, rows, sem):
        pltpu.async_copy(hl.at[dsrc], rows, sem)

    def wait_gather(rows, sem):
        pltpu.make_async_copy(hl.at[pl.ds(0, _KG)], rows, sem).wait()

    def accum(ddst, dew, rows, nval):
        nsub = nval >> 4

        def _sub(g, c3):
            gb = g * 16
            dv16 = ddst[pl.ds(gb, 16)]
            wv16 = dew[pl.ds(gb, 16)]
            for k in range(16):
                li = dv16[k] - base
                w = wv16[k]
                for v in range(_NV):
                    val = rows[gb + k, pl.ds(v * 16, 16)] * w
                    plsc.addupdate(acc.at[li, pl.ds(v * 16, 16)], val)
            return c3

        lax.fori_loop(0, nsub, _sub, 0)

        def _edge(e, c3):
            li = ddst[pl.ds(e, 16)][0] - base
            w = dew[pl.ds(e, 16)][0]
            for v in range(_NV):
                val = rows[e, pl.ds(v * 16, 16)] * w
                plsc.addupdate(acc.at[li, pl.ds(v * 16, 16)], val)
            return c3

        lax.fori_loop(nsub * 16, nval, _edge, 0)

    def extras(s, rows, semG):
        off, cnt = meta(s)
        nch = ((cnt + _KG - 1) * 23832) >> 21

        def _ch(ch, c2):
            fire_stage(s, ch, sC_src, sC_dst, sC_ew, semSC)
            wait_stage(sC_src, sC_dst, sC_ew, semSC)
            fire_gather(sC_src, rows, semG)
            wait_gather(rows, semG)
            accum(sC_dst, sC_ew, rows, jnp.minimum(cnt - ch * _KG, _KG))
            return c2

        lax.fori_loop(1, nch, _ch, 0)

    fire_stage(0, 0, sA_src, sA_dst, sA_ew, semSA)
    wait_stage(sA_src, sA_dst, sA_ew, semSA)
    fire_gather(sA_src, rowsA, semGA)
    fire_stage(1, 0, sB_src, sB_dst, sB_ew, semSB)

    def body(p, c):
        s0 = p * 2
        s1 = s0 + 1
        wait_gather(rowsA, semGA)
        wait_stage(sB_src, sB_dst, sB_ew, semSB)
        fire_gather(sB_src, rowsB, semGB)
        fire_stage(s0 + 2, 0, sA_src, sA_dst, sA_ew, semSA)
        _, cnt0 = meta(s0)
        accum(sA_dst, sA_ew, rowsA, jnp.minimum(cnt0, _KG))
        extras(s0, rowsA, semGA)
        wait_gather(rowsB, semGB)
        wait_stage(sA_src, sA_dst, sA_ew, semSA)
        fire_gather(sA_src, rowsA, semGA)
        fire_stage(s1 + 2, 0, sB_src, sB_dst, sB_ew, semSB)
        _, cnt1 = meta(s1)
        accum(sB_dst, sB_ew, rowsB, jnp.minimum(cnt1, _KG))
        extras(s1, rowsB, semGB)
        return c

    lax.fori_loop(0, _NT // 2, body, 0)
    wait_gather(rowsA, semGA)
    wait_stage(sB_src, sB_dst, sB_ew, semSB)

    pltpu.sync_copy(acc, out.at[pl.ds(base, _NPT)])


_sc_acc = functools.partial(
    pl.kernel,
    out_type=jax.ShapeDtypeStruct((_NPAD, _D), jnp.float32),
    mesh=plsc.VectorSubcoreMesh(core_axis_name="c", subcore_axis_name="s"),
    scratch_types=[
        pltpu.VMEM((_NPT, _D), jnp.float32),
        pltpu.VMEM((_NT * 64 + 16,), jnp.int32),
        pltpu.VMEM((_KG,), jnp.int32),
        pltpu.VMEM((_KG + 16,), jnp.int32),
        pltpu.VMEM((_KG + 16,), jnp.float32),
        pltpu.VMEM((_KG,), jnp.int32),
        pltpu.VMEM((_KG + 16,), jnp.int32),
        pltpu.VMEM((_KG + 16,), jnp.float32),
        pltpu.VMEM((_KG,), jnp.int32),
        pltpu.VMEM((_KG + 16,), jnp.int32),
        pltpu.VMEM((_KG + 16,), jnp.float32),
        pltpu.VMEM((_KG, _D), jnp.float32),
        pltpu.VMEM((_KG, _D), jnp.float32),
        pltpu.SemaphoreType.DMA,
        pltpu.SemaphoreType.DMA,
        pltpu.SemaphoreType.DMA,
        pltpu.SemaphoreType.DMA,
        pltpu.SemaphoreType.DMA,
    ],
)(_acc_body)


def _mm_body(x_ref, w_ref, o_ref):
    o_ref[...] = jnp.dot(x_ref[...], w_ref[...],
                         preferred_element_type=jnp.float32)


def _bn_mm_body(agg_ref, b_ref, g_ref, be_ref, w_ref, h_ref, hl_ref):
    h = agg_ref[...] + b_ref[...]
    mu = jnp.mean(h, axis=0, keepdims=True)
    d = h - mu
    var = jnp.mean(d * d, axis=0, keepdims=True)
    hr = jnp.maximum(d * lax.rsqrt(var + 1e-5) * g_ref[...] + be_ref[...], 0.0)
    h_ref[...] = hr
    hl_ref[...] = jnp.dot(hr, w_ref[...], preferred_element_type=jnp.float32)


def _final_body(agg_ref, b_ref, g_ref, be_ref, x_ref, h1_ref, bf_ref,
                wp0_ref, bp0_ref, wp1_ref, bp1_ref, wp2_ref, bp2_ref, o_ref):
    h = agg_ref[...] + b_ref[...]
    mu = jnp.mean(h, axis=0, keepdims=True)
    d = h - mu
    var = jnp.mean(d * d, axis=0, keepdims=True)
    h2 = jnp.maximum(d * lax.rsqrt(var + 1e-5) * g_ref[...] + be_ref[...], 0.0)

    ids = lax.broadcasted_iota(jnp.int32, (_G, _N), 0)
    oh = (bf_ref[...] == ids).astype(jnp.float32)
    counts = jnp.maximum(jnp.sum(oh, axis=1, keepdims=True), 1.0)
    p0 = jnp.dot(oh, x_ref[...], preferred_element_type=jnp.float32) / counts
    p1 = jnp.dot(oh, h1_ref[...], preferred_element_type=jnp.float32) / counts
    p2 = jnp.dot(oh, h2, preferred_element_type=jnp.float32) / counts
    r = (jnp.dot(p0, wp0_ref[...], preferred_element_type=jnp.float32)
         + bp0_ref[...]
         + jnp.dot(p1, wp1_ref[...], preferred_element_type=jnp.float32)
         + bp1_ref[...]
         + jnp.dot(p2, wp2_ref[...], preferred_element_type=jnp.float32)
         + bp2_ref[...])
    o_ref[...] = jax.nn.sigmoid(r)


def kernel(x, edge_index, edge_weight, batch,
           W0, b0, W1, b1, g0, be0, g1, be1,
           Wp0, bp0, Wp1, bp1, Wp2, bp2):
    src = edge_index[0]
    dst = edge_index[1]
    batch_f = batch.reshape(1, _N)

    b0r, g0r, be0r = (v.reshape(1, _D) for v in (b0, g0, be0))
    b1r, g1r, be1r = (v.reshape(1, _D) for v in (b1, g1, be1))
    bp0r, bp1r, bp2r = (v.reshape(1, _DO) for v in (bp0, bp1, bp2))

    lsrc, ldst, lew, ltab = _sc_filter(src, dst, edge_weight)

    hl0 = pl.pallas_call(
        _mm_body,
        out_shape=jax.ShapeDtypeStruct((_N, _D), jnp.float32),
    )(x, W0)

    agg0 = _sc_acc(hl0, lsrc, ldst, lew, ltab)[:_N]

    h1, hl1 = pl.pallas_call(
        _bn_mm_body,
        out_shape=(jax.ShapeDtypeStruct((_N, _D), jnp.float32),
                   jax.ShapeDtypeStruct((_N, _D), jnp.float32)),
    )(agg0, b0r, g0r, be0r, W1)

    agg1 = _sc_acc(hl1, lsrc, ldst, lew, ltab)[:_N]

    out = pl.pallas_call(
        _final_body,
        out_shape=jax.ShapeDtypeStruct((_G, _DO), jnp.float32),
    )(agg1, b1r, g1r, be1r, x, h1, batch_f,
      Wp0, bp0r, Wp1, bp1r, Wp2, bp2r)
    return out
